# baseline (speedup 1.0000x reference)
import jax
import jax.numpy as jnp
from jax import lax
from jax.experimental import pallas as pl
from jax.experimental.pallas import tpu as pltpu
from jax.experimental.pallas import tpu_sc as plsc

B, P, F, N = 1024, 100, 128, 512
NF = 6 * F
ROWS = B * P
NW = 32
RPW = ROWS // NW
CH = 128
NCH = RPW // CH
L = 16


def _body(f0, f1, f2, f3, f4, idx_hbm, emb_hbm, out_hbm,
          idx_raw, idx_g, gbuf, b0, b1, b2, b3, b4, gsem, rsem, wsem):
    wid = lax.axis_index("s") * 2 + lax.axis_index("c")
    r0w = wid * RPW
    fbufs = (b0, b1, b2, b3, b4)

    @pl.loop(0, NCH)
    def _chunk(c):
        r0 = r0w + c * CH
        pltpu.sync_copy(idx_hbm.at[pl.ds(r0, CH)], idx_raw)
        r0v = jnp.full((L,), r0, jnp.int32)
        for j in range(CH // L):
            rows = r0v + (lax.iota(jnp.int32, L) + jnp.full((L,), j * L, jnp.int32))
            b = lax.div(rows, jnp.full((L,), P, jnp.int32))
            idx_g[pl.ds(j * L, L)] = idx_raw[pl.ds(j * L, L)] + b * jnp.full((L,), N, jnp.int32)
        gd = pltpu.async_copy(emb_hbm.at[idx_g], gbuf, gsem)
        rd = [pltpu.async_copy(f.at[pl.ds(r0, CH), :], fb, rsem)
              for f, fb in zip((f0, f1, f2, f3, f4), fbufs)]
        for d in rd:
            d.wait()
        gd.wait()
        wr = [pltpu.async_copy(fb, out_hbm.at[pl.ds(r0, CH), pl.ds(k * F, F)], wsem)
              for k, fb in enumerate(fbufs)]
        wr.append(pltpu.async_copy(gbuf, out_hbm.at[pl.ds(r0, CH), pl.ds(5 * F, F)], wsem))
        for d in wr:
            d.wait()


def kernel(known_one_hot, unknown_one_hot, known_differ_one_hot,
           workers_qa_turn_one_hot, workers_max_qa_turn_one_hot,
           personal_nodes, final_node_embed):
    feats = [x.reshape(ROWS, F) for x in
             (known_one_hot, unknown_one_hot, known_differ_one_hot,
              workers_qa_turn_one_hot, workers_max_qa_turn_one_hot)]
    idx = personal_nodes.reshape(ROWS).astype(jnp.int32)
    emb = final_node_embed.reshape(B * N, F)

    mesh = plsc.VectorSubcoreMesh(core_axis_name="c", subcore_axis_name="s")
    run = pl.kernel(
        _body,
        out_type=jax.ShapeDtypeStruct((ROWS, NF), jnp.float32),
        mesh=mesh,
        scratch_types=[
            pltpu.VMEM((CH,), jnp.int32),
            pltpu.VMEM((CH,), jnp.int32),
            pltpu.VMEM((CH, F), jnp.float32),
            pltpu.VMEM((CH, F), jnp.float32),
            pltpu.VMEM((CH, F), jnp.float32),
            pltpu.VMEM((CH, F), jnp.float32),
            pltpu.VMEM((CH, F), jnp.float32),
            pltpu.VMEM((CH, F), jnp.float32),
            pltpu.SemaphoreType.DMA,
            pltpu.SemaphoreType.DMA,
            pltpu.SemaphoreType.DMA,
        ],
    )
    out = run(*feats, idx, emb)
    return out.reshape(B, P, NF)


# speedup vs baseline: 1.6924x; 1.6924x over previous
import jax
import jax.numpy as jnp
from jax import lax
from jax.experimental import pallas as pl
from jax.experimental.pallas import tpu as pltpu
from jax.experimental.pallas import tpu_sc as plsc

B, P, F, N = 1024, 100, 128, 512
NF = 6 * F
NW = 32
BPW = B // NW


def _body(f0, f1, f2, f3, f4, idx_hbm, emb_hbm, out_hbm,
          idx_v, gbuf, b0, b1, b2, b3, b4, gsem, rsem, wsem):
    wid = lax.axis_index("s") * 2 + lax.axis_index("c")
    base = wid * BPW
    fbufs = (b0, b1, b2, b3, b4)

    @pl.loop(0, BPW)
    def _batch(i):
        b = base + i
        pltpu.sync_copy(idx_hbm.at[b], idx_v)
        gd = pltpu.async_copy(emb_hbm.at[b].at[idx_v], gbuf, gsem)
        rds = [pltpu.async_copy(f.at[b], fb, rsem)
               for f, fb in zip((f0, f1, f2, f3, f4), fbufs)]
        wrs = []
        for k, fb in enumerate(fbufs):
            rds[k].wait()
            wrs.append(pltpu.async_copy(
                fb, out_hbm.at[b, :, pl.ds(k * F, F)], wsem))
        gd.wait()
        wrs.append(pltpu.async_copy(
            gbuf, out_hbm.at[b, :, pl.ds(5 * F, F)], wsem))
        for d in wrs:
            d.wait()


def kernel(known_one_hot, unknown_one_hot, known_differ_one_hot,
           workers_qa_turn_one_hot, workers_max_qa_turn_one_hot,
           personal_nodes, final_node_embed):
    idx = personal_nodes.astype(jnp.int32)

    mesh = plsc.VectorSubcoreMesh(core_axis_name="c", subcore_axis_name="s")
    run = pl.kernel(
        _body,
        out_type=jax.ShapeDtypeStruct((B, P, NF), jnp.float32),
        mesh=mesh,
        compiler_params=pltpu.CompilerParams(use_tc_tiling_on_sc=True),
        scratch_types=[
            pltpu.VMEM((P,), jnp.int32),
            pltpu.VMEM((P, F), jnp.float32),
            pltpu.VMEM((P, F), jnp.float32),
            pltpu.VMEM((P, F), jnp.float32),
            pltpu.VMEM((P, F), jnp.float32),
            pltpu.VMEM((P, F), jnp.float32),
            pltpu.VMEM((P, F), jnp.float32),
            pltpu.SemaphoreType.DMA,
            pltpu.SemaphoreType.DMA,
            pltpu.SemaphoreType.DMA,
        ],
    )
    out = run(known_one_hot, unknown_one_hot, known_differ_one_hot,
              workers_qa_turn_one_hot, workers_max_qa_turn_one_hot,
              idx, final_node_embed)
    return out


# speedup vs baseline: 1.6983x; 1.0035x over previous
import jax
import jax.numpy as jnp
from jax import lax
from jax.experimental import pallas as pl
from jax.experimental.pallas import tpu as pltpu
from jax.experimental.pallas import tpu_sc as plsc

B, P, F, N = 1024, 100, 128, 512
NF = 6 * F
PP = 104
NW = 32
BPW = B // NW
BB = 8


def _gather_body(idx_hbm, emb_hbm, out_hbm, idx_v, gbuf, gsem):
    wid = lax.axis_index("s") * 2 + lax.axis_index("c")
    base = wid * BPW

    @pl.loop(0, BPW)
    def _batch(i):
        b = base + i
        pltpu.sync_copy(idx_hbm.at[b], idx_v)
        pltpu.async_copy(emb_hbm.at[b].at[idx_v.at[pl.ds(0, PP)]], gbuf, gsem).wait()
        pltpu.sync_copy(gbuf, out_hbm.at[b])


def _concat_body(f0, f1, f2, f3, f4, g, out):
    for k, f in enumerate((f0, f1, f2, f3, f4)):
        out[:, :, pl.ds(k * F, F)] = f[...]
    out[:, :, pl.ds(5 * F, F)] = g[:, :P, :]


def kernel(known_one_hot, unknown_one_hot, known_differ_one_hot,
           workers_qa_turn_one_hot, workers_max_qa_turn_one_hot,
           personal_nodes, final_node_embed):
    idx = jnp.pad(personal_nodes.astype(jnp.int32), ((0, 0), (0, 128 - P)))

    mesh = plsc.VectorSubcoreMesh(core_axis_name="c", subcore_axis_name="s")
    gathered = pl.kernel(
        _gather_body,
        out_type=jax.ShapeDtypeStruct((B, PP, F), jnp.float32),
        mesh=mesh,
        compiler_params=pltpu.CompilerParams(use_tc_tiling_on_sc=True),
        scratch_types=[
            pltpu.VMEM((128,), jnp.int32),
            pltpu.VMEM((PP, F), jnp.float32),
            pltpu.SemaphoreType.DMA,
        ],
    )(idx, final_node_embed)

    feat_spec = pl.BlockSpec((BB, P, F), lambda i: (i, 0, 0))
    out = pl.pallas_call(
        _concat_body,
        grid=(B // BB,),
        in_specs=[feat_spec] * 5 + [pl.BlockSpec((BB, PP, F), lambda i: (i, 0, 0))],
        out_specs=pl.BlockSpec((BB, P, NF), lambda i: (i, 0, 0)),
        out_shape=jax.ShapeDtypeStruct((B, P, NF), jnp.float32),
    )(known_one_hot, unknown_one_hot, known_differ_one_hot,
      workers_qa_turn_one_hot, workers_max_qa_turn_one_hot, gathered)
    return out


# speedup vs baseline: 2.2207x; 1.3076x over previous
import jax
import jax.numpy as jnp
from jax import lax
from jax.experimental import pallas as pl
from jax.experimental.pallas import tpu as pltpu
from jax.experimental.pallas import tpu_sc as plsc

B, P, F, N = 1024, 100, 128, 512
PP = 104
NW = 32
BPW = B // NW


def _gather_body(idx_hbm, emb_hbm, out_hbm, idx_all, gbuf, gsem):
    wid = lax.axis_index("s") * 2 + lax.axis_index("c")
    base = wid * BPW
    pltpu.sync_copy(idx_hbm.at[pl.ds(base, BPW), :], idx_all)

    @pl.loop(0, BPW)
    def _batch(i):
        b = base + i
        pltpu.async_copy(
            emb_hbm.at[b].at[idx_all.at[i, pl.ds(0, PP)]], gbuf, gsem).wait()
        pltpu.sync_copy(gbuf, out_hbm.at[b])


def kernel(known_one_hot, unknown_one_hot, known_differ_one_hot,
           workers_qa_turn_one_hot, workers_max_qa_turn_one_hot,
           personal_nodes, final_node_embed):
    idx = jnp.pad(personal_nodes.astype(jnp.int32), ((0, 0), (0, 128 - P)))

    mesh = plsc.VectorSubcoreMesh(core_axis_name="c", subcore_axis_name="s")
    gathered = pl.kernel(
        _gather_body,
        out_type=jax.ShapeDtypeStruct((B, PP, F), jnp.float32),
        mesh=mesh,
        compiler_params=pltpu.CompilerParams(use_tc_tiling_on_sc=True),
        scratch_types=[
            pltpu.VMEM((BPW, 128), jnp.int32),
            pltpu.VMEM((PP, F), jnp.float32),
            pltpu.SemaphoreType.DMA,
        ],
    )(idx, final_node_embed)

    return jnp.concatenate(
        (known_one_hot, unknown_one_hot, known_differ_one_hot,
         workers_qa_turn_one_hot, workers_max_qa_turn_one_hot,
         gathered[:, :P, :]), axis=2)


# speedup vs baseline: 2.2241x; 1.0015x over previous
import jax
import jax.numpy as jnp
from jax import lax
from jax.experimental import pallas as pl
from jax.experimental.pallas import tpu as pltpu
from jax.experimental.pallas import tpu_sc as plsc

B, P, F, N = 1024, 100, 128, 512
PP = 104
NW = 32
BPW = B // NW


def _gather_body(idx_hbm, emb_hbm, out_hbm, idx_all, gbuf, gsem):
    wid = lax.axis_index("s") * 2 + lax.axis_index("c")
    base = wid * BPW
    pltpu.sync_copy(idx_hbm.at[pl.ds(base, BPW), :], idx_all)

    @pl.loop(0, BPW)
    def _batch(i):
        b = base + i
        pltpu.async_copy(
            emb_hbm.at[b].at[idx_all.at[i, pl.ds(0, PP)]], gbuf, gsem).wait()
        pltpu.sync_copy(gbuf, out_hbm.at[b])


def kernel(known_one_hot, unknown_one_hot, known_differ_one_hot,
           workers_qa_turn_one_hot, workers_max_qa_turn_one_hot,
           personal_nodes, final_node_embed):
    idx = jnp.pad(personal_nodes.astype(jnp.int32), ((0, 0), (0, 128 - P)))

    mesh = plsc.VectorSubcoreMesh(core_axis_name="c", subcore_axis_name="s")
    gathered = pl.kernel(
        _gather_body,
        out_type=jax.ShapeDtypeStruct((B, PP, F), jnp.float32),
        mesh=mesh,
        compiler_params=pltpu.CompilerParams(use_tc_tiling_on_sc=True),
        scratch_types=[
            pltpu.VMEM((BPW, 128), jnp.int32),
            pltpu.VMEM((PP, F), jnp.float32),
            pltpu.SemaphoreType.DMA,
        ],
    )(idx, final_node_embed)

    gathered = lax.optimization_barrier(gathered[:, :P, :])
    return jnp.concatenate(
        (known_one_hot, unknown_one_hot, known_differ_one_hot,
         workers_qa_turn_one_hot, workers_max_qa_turn_one_hot,
         gathered), axis=2)
